# Initial kernel scaffold; baseline (speedup 1.0000x reference)
import jax
import jax.numpy as jnp
from jax.experimental import pallas as pl


def kernel(x, token_table, pos_table):
    raise NotImplementedError("write your pallas kernel here")


# speedup vs baseline: 3.4016x; 3.4016x over previous
import functools

import jax
import jax.numpy as jnp
from jax import lax
from jax.experimental import pallas as pl
from jax.experimental.pallas import tpu as pltpu
from jax.experimental.pallas import tpu_sc as plsc

VOCAB = 100000
MAXLEN = 200
EMBED = 32
BATCH = 4096

NC = 2
NS = 16
NW = NC * NS
ROWS = BATCH * MAXLEN
RPW = ROWS // NW
C = 1024
GSUB = 128
NSUB = C // GSUB
NCHUNK = RPW // C
PWORDS = MAXLEN * EMBED
POS_BIG = C * EMBED + PWORDS

_mesh = plsc.VectorSubcoreMesh(core_axis_name="c", subcore_axis_name="s")


@functools.partial(
    pl.kernel,
    mesh=_mesh,
    compiler_params=pltpu.CompilerParams(use_tc_tiling_on_sc=False),
    out_type=jax.ShapeDtypeStruct((ROWS, EMBED), jnp.float32),
    scratch_types=[
        pltpu.VMEM((NSUB, GSUB), jnp.int32),
        pltpu.VMEM((C, EMBED), jnp.float32),
        pltpu.VMEM((POS_BIG,), jnp.float32),
        pltpu.SemaphoreType.DMA,
    ],
)
def _emb(idx_hbm, tok_hbm, pos_rep_hbm, out_hbm, idx_v, rows_v, pos_v, gsem):
    wid = lax.axis_index("s") * NC + lax.axis_index("c")
    base = wid * RPW
    pltpu.sync_copy(pos_rep_hbm, pos_v)

    def chunk_body(g, carry):
        b0 = base + g * C
        p0 = lax.rem(g * C, MAXLEN)
        pltpu.sync_copy(
            idx_hbm.at[pl.ds(pl.multiple_of(b0 // GSUB, 8), NSUB)], idx_v
        )
        handles = []
        for j in range(NSUB):
            handles.append(
                pltpu.async_copy(
                    tok_hbm.at[idx_v.at[j]],
                    rows_v.at[pl.ds(j * GSUB, GSUB)],
                    gsem,
                )
            )
        for h in handles:
            h.wait()

        def add_body(r, p):
            off = (p + r) * EMBED
            rows_v[r, pl.ds(0, 16)] += pos_v[pl.ds(off, 16)]
            rows_v[r, pl.ds(16, 16)] += pos_v[pl.ds(off + 16, 16)]
            return p

        lax.fori_loop(0, C, add_body, p0)
        pltpu.sync_copy(rows_v, out_hbm.at[pl.ds(b0, C)])
        return carry

    lax.fori_loop(0, NCHUNK, chunk_body, 0)


def kernel(x, token_table, pos_table):
    idx = x.reshape(ROWS // GSUB, GSUB).astype(jnp.int32)
    pos_flat = pos_table.reshape(-1)
    reps = -(-POS_BIG // PWORDS)
    pos_rep = jnp.tile(pos_flat, reps)[:POS_BIG]
    out = _emb(idx, token_table, pos_rep)
    return out.reshape(BATCH, MAXLEN, EMBED)
